# baseline (speedup 1.0000x reference)
import functools

import jax
import jax.numpy as jnp
from jax.experimental import pallas as pl
from jax.experimental.pallas import tpu as pltpu
from jax.experimental.pallas import tpu_sc as plsc

_NUM_LAYERS = 12
_W = 3 + 2 * _NUM_LAYERS + 3 * _NUM_LAYERS + 3 + 2


def _sc_lookup(x, tab):
    mesh = plsc.ScalarSubcoreMesh(axis_name="c", num_cores=1)

    @functools.partial(
        pl.kernel,
        mesh=mesh,
        compiler_params=pltpu.CompilerParams(use_tc_tiling_on_sc=False),
        out_type=jax.ShapeDtypeStruct((_W,), jnp.float32),
        scratch_types=(
            pltpu.SMEM((1,), jnp.float32),
        ),
    )
    def k(x_hbm, t, o, x_s):
        pltpu.sync_copy(x_hbm, x_s)
        idx = (x_s[0] * 101.0).astype(jnp.int32)
        pltpu.sync_copy(t.at[idx], o)

    return k(x.reshape(1), tab)


def kernel(x, E_layer, E_head, E_mlp, E_embed, E_bias):
    tab = jnp.concatenate([E_layer, E_head, E_mlp, E_embed, E_bias], axis=1)
    row = _sc_lookup(x, tab)
    return (
        row[0:3],
        row[3:27].reshape(_NUM_LAYERS, 2),
        row[27:63].reshape(_NUM_LAYERS, 3),
        row[63:66],
        row[66:68],
    )
